# baseline (speedup 1.0000x reference)
import functools

import jax
import jax.numpy as jnp
from jax import lax
from jax.experimental import pallas as pl
from jax.experimental.pallas import tpu as pltpu
from jax.experimental.pallas import tpu_sc as plsc

_EPS = 1e-5

_NC = 2
_NS = 16
_NW = _NC * _NS
_CHUNK = 128


def _sc_gather_body(uidx_hbm, iidx_hbm, uemb_hbm, iemb_hbm, ulin_hbm, ilin_hbm,
                    uemb_out, iemb_out, ulin_out, ilin_out,
                    uidx_v, iidx_v, uemb_v, iemb_v, ulin_v, ilin_v, sem):
    n_chunks = uidx_v.shape[0]
    b_per_w = n_chunks * _CHUNK
    wid = lax.axis_index("s") * _NC + lax.axis_index("c")
    base = wid * b_per_w

    pltpu.sync_copy(uidx_hbm.at[wid], uidx_v)
    pltpu.sync_copy(iidx_hbm.at[wid], iidx_v)

    copies = []
    for c in range(n_chunks):
        sl = pl.ds(c * _CHUNK, _CHUNK)
        copies.append(pltpu.async_copy(uemb_hbm.at[uidx_v.at[c]], uemb_v.at[sl], sem))
        copies.append(pltpu.async_copy(iemb_hbm.at[iidx_v.at[c]], iemb_v.at[sl], sem))
        copies.append(pltpu.async_copy(ulin_hbm.at[uidx_v.at[c]], ulin_v.at[sl], sem))
        copies.append(pltpu.async_copy(ilin_hbm.at[iidx_v.at[c]], ilin_v.at[sl], sem))
    for cp in copies:
        cp.wait()

    out_sl = pl.ds(base, b_per_w)
    pltpu.sync_copy(uemb_v, uemb_out.at[out_sl])
    pltpu.sync_copy(iemb_v, iemb_out.at[out_sl])
    pltpu.sync_copy(ulin_v, ulin_out.at[out_sl])
    pltpu.sync_copy(ilin_v, ilin_out.at[out_sl])


def _sc_gather(user_ids, item_ids, user_emb_table, item_emb_table,
               user_lin_table, item_lin_table):
    batch = user_ids.shape[0]
    emb_dim = user_emb_table.shape[1]
    b_per_w = batch // _NW
    n_chunks = b_per_w // _CHUNK
    uidx = user_ids.reshape(_NW, n_chunks, _CHUNK)
    iidx = item_ids.reshape(_NW, n_chunks, _CHUNK)

    mesh = plsc.VectorSubcoreMesh(core_axis_name="c", subcore_axis_name="s")
    f32 = jnp.float32
    run = pl.kernel(
        _sc_gather_body,
        out_type=(
            jax.ShapeDtypeStruct((batch, emb_dim), f32),
            jax.ShapeDtypeStruct((batch, emb_dim), f32),
            jax.ShapeDtypeStruct((batch, 1), f32),
            jax.ShapeDtypeStruct((batch, 1), f32),
        ),
        mesh=mesh,
        compiler_params=pltpu.CompilerParams(use_tc_tiling_on_sc=False),
        scratch_types=(
            pltpu.VMEM((n_chunks, _CHUNK), jnp.int32),
            pltpu.VMEM((n_chunks, _CHUNK), jnp.int32),
            pltpu.VMEM((b_per_w, emb_dim), f32),
            pltpu.VMEM((b_per_w, emb_dim), f32),
            pltpu.VMEM((b_per_w, 1), f32),
            pltpu.VMEM((b_per_w, 1), f32),
            pltpu.SemaphoreType.DMA,
        ),
    )
    return run(uidx, iidx, user_emb_table, item_emb_table,
               user_lin_table, item_lin_table)


def _tc_mlp_body(uemb_ref, iemb_ref, ulin_ref, ilin_ref, bias_ref,
                 w1u_ref, w1i_ref, b1_ref, g1_ref, be1_ref,
                 w2_ref, b2_ref, g2_ref, be2_ref,
                 w3_ref, b3_ref, out_ref):
    u = uemb_ref[...]
    it = iemb_ref[...]
    batch = u.shape[0]

    fm = jnp.sum(u * it, axis=1, keepdims=True)
    lin = ulin_ref[...] + ilin_ref[...] + bias_ref[0, 0]

    h = (jnp.dot(u, w1u_ref[...], preferred_element_type=jnp.float32)
         + jnp.dot(it, w1i_ref[...], preferred_element_type=jnp.float32)
         + b1_ref[...])
    m1 = jnp.mean(h, axis=0, keepdims=True)
    v1 = jnp.mean((h - m1) * (h - m1), axis=0, keepdims=True)
    h = g1_ref[...] * (h - m1) * lax.rsqrt(v1 + _EPS) + be1_ref[...]
    h = jnp.maximum(h, 0.0)

    h2 = jnp.dot(h, w2_ref[...], preferred_element_type=jnp.float32) + b2_ref[...]
    m2 = jnp.mean(h2, axis=0, keepdims=True)
    v2 = jnp.mean((h2 - m2) * (h2 - m2), axis=0, keepdims=True)
    h2 = g2_ref[...] * (h2 - m2) * lax.rsqrt(v2 + _EPS) + be2_ref[...]
    h2 = jnp.maximum(h2, 0.0)

    deep = jnp.sum(h2 * w3_ref[...], axis=1, keepdims=True) + b3_ref[0, 0]

    out_ref[...] = jax.nn.sigmoid(lin + fm + deep)


def _tc_mlp(uemb, iemb, ulin, ilin, bias, W1, b1, g1, be1,
            W2, b2, g2, be2, W3, b3):
    batch = uemb.shape[0]
    emb_dim = uemb.shape[1]
    w1u = W1[:, :emb_dim].T
    w1i = W1[:, emb_dim:].T
    w2 = W2.T
    return pl.pallas_call(
        _tc_mlp_body,
        out_shape=jax.ShapeDtypeStruct((batch, 1), jnp.float32),
        compiler_params=pltpu.CompilerParams(
            vmem_limit_bytes=100 * 1024 * 1024,
        ),
    )(uemb, iemb, ulin, ilin, bias.reshape(1, 1),
      w1u, w1i, b1.reshape(1, -1), g1.reshape(1, -1), be1.reshape(1, -1),
      w2, b2.reshape(1, -1), g2.reshape(1, -1), be2.reshape(1, -1),
      W3, b3.reshape(1, 1))


def kernel(user_ids, item_ids, user_emb_table, item_emb_table,
           user_lin_table, item_lin_table, bias,
           W1, b1, g1, be1, W2, b2, g2, be2, W3, b3):
    uemb, iemb, ulin, ilin = _sc_gather(
        user_ids, item_ids, user_emb_table, item_emb_table,
        user_lin_table, item_lin_table)
    pred = _tc_mlp(uemb, iemb, ulin, ilin, bias, W1, b1, g1, be1,
                   W2, b2, g2, be2, W3, b3)
    return jnp.squeeze(pred, axis=-1)


# speedup vs baseline: 1.1138x; 1.1138x over previous
import jax
import jax.numpy as jnp
from jax import lax
from jax.experimental import pallas as pl
from jax.experimental.pallas import tpu as pltpu
from jax.experimental.pallas import tpu_sc as plsc

_EPS = 1e-5

_NC = 2
_NS = 16
_NW = _NC * _NS
_CHUNK = 128

_PACK_BLK = 256


def _pack_half(v):
    return (((v + 1) // 2 + _PACK_BLK - 1) // _PACK_BLK) * _PACK_BLK


def _pack_body(x1_ref, x2_ref, o_ref):
    o_ref[...] = jnp.concatenate([x1_ref[...].T, x2_ref[...].T], axis=1)


def _tc_pack(xt):
    d, v = xt.shape
    half = _pack_half(v)
    n_blocks = half // _PACK_BLK
    last_blk = (v - 1) // _PACK_BLK
    return pl.pallas_call(
        _pack_body,
        grid=(n_blocks,),
        in_specs=[
            pl.BlockSpec((d, _PACK_BLK), lambda j: (0, j)),
            pl.BlockSpec((d, _PACK_BLK),
                         lambda j: (0, jnp.minimum(j + n_blocks, last_blk))),
        ],
        out_specs=pl.BlockSpec((_PACK_BLK, 128), lambda j: (j, 0)),
        out_shape=jax.ShapeDtypeStruct((half, 128), jnp.float32),
    )(xt, xt)


def _sc_gather_body(uidx_hbm, iidx_hbm, pu_hbm, pi_hbm,
                    ug_out, ig_out, idx_v, rows_v, sem):
    n_chunks = idx_v.shape[0]
    b_per_w = n_chunks * _CHUNK
    wid = lax.axis_index("s") * _NC + lax.axis_index("c")
    base = wid * b_per_w
    out_sl = pl.ds(base, b_per_w)

    pltpu.sync_copy(uidx_hbm.at[wid], idx_v)
    copies = [pltpu.async_copy(pu_hbm.at[idx_v.at[c]],
                               rows_v.at[pl.ds(c * _CHUNK, _CHUNK)], sem)
              for c in range(n_chunks)]
    for cp in copies:
        cp.wait()
    pltpu.sync_copy(rows_v, ug_out.at[out_sl])

    pltpu.sync_copy(iidx_hbm.at[wid], idx_v)
    copies = [pltpu.async_copy(pi_hbm.at[idx_v.at[c]],
                               rows_v.at[pl.ds(c * _CHUNK, _CHUNK)], sem)
              for c in range(n_chunks)]
    for cp in copies:
        cp.wait()
    pltpu.sync_copy(rows_v, ig_out.at[out_sl])


def _sc_gather_packed(upidx, ipidx, pu, pi):
    batch = upidx.size
    b_per_w = batch // _NW
    n_chunks = b_per_w // _CHUNK
    uidx = upidx.reshape(_NW, n_chunks, _CHUNK)
    iidx = ipidx.reshape(_NW, n_chunks, _CHUNK)

    mesh = plsc.VectorSubcoreMesh(core_axis_name="c", subcore_axis_name="s")
    run = pl.kernel(
        _sc_gather_body,
        out_type=(
            jax.ShapeDtypeStruct((batch, 128), jnp.float32),
            jax.ShapeDtypeStruct((batch, 128), jnp.float32),
        ),
        mesh=mesh,
        compiler_params=pltpu.CompilerParams(use_tc_tiling_on_sc=True),
        scratch_types=(
            pltpu.VMEM((n_chunks, _CHUNK), jnp.int32),
            pltpu.VMEM((b_per_w, 128), jnp.float32),
            pltpu.SemaphoreType.DMA,
        ),
    )
    return run(uidx, iidx, pu, pi)


def _sc_lin_body(uidx_hbm, iidx_hbm, ulin_hbm, ilin_hbm,
                 ulin_out, ilin_out, idx_v, val_u, val_i, sem):
    n_chunks = idx_v.shape[0]
    b_per_w = n_chunks * _CHUNK
    wid = lax.axis_index("s") * _NC + lax.axis_index("c")
    out_sl = pl.ds(wid * b_per_w, b_per_w)

    pltpu.sync_copy(uidx_hbm.at[wid], idx_v)
    copies = [pltpu.async_copy(ulin_hbm.at[idx_v.at[c]],
                               val_u.at[pl.ds(c * _CHUNK, _CHUNK)], sem)
              for c in range(n_chunks)]
    for cp in copies:
        cp.wait()
    pltpu.sync_copy(iidx_hbm.at[wid], idx_v)
    copies = [pltpu.async_copy(ilin_hbm.at[idx_v.at[c]],
                               val_i.at[pl.ds(c * _CHUNK, _CHUNK)], sem)
              for c in range(n_chunks)]
    for cp in copies:
        cp.wait()
    pltpu.sync_copy(val_u, ulin_out.at[out_sl])
    pltpu.sync_copy(val_i, ilin_out.at[out_sl])


def _sc_lin(user_ids, item_ids, ulin1, ilin1):
    batch = user_ids.shape[0]
    b_per_w = batch // _NW
    n_chunks = b_per_w // _CHUNK
    uidx = user_ids.reshape(_NW, n_chunks, _CHUNK)
    iidx = item_ids.reshape(_NW, n_chunks, _CHUNK)

    mesh = plsc.VectorSubcoreMesh(core_axis_name="c", subcore_axis_name="s")
    run = pl.kernel(
        _sc_lin_body,
        out_type=(
            jax.ShapeDtypeStruct((batch,), jnp.float32),
            jax.ShapeDtypeStruct((batch,), jnp.float32),
        ),
        mesh=mesh,
        compiler_params=pltpu.CompilerParams(use_tc_tiling_on_sc=False),
        scratch_types=(
            pltpu.VMEM((n_chunks, _CHUNK), jnp.int32),
            pltpu.VMEM((b_per_w,), jnp.float32),
            pltpu.VMEM((b_per_w,), jnp.float32),
            pltpu.SemaphoreType.DMA,
        ),
    )
    return run(uidx, iidx, ulin1, ilin1)


_M1_BLK = 2048


def _tc_mlp1_body(ug_ref, ig_ref, upar_ref, ipar_ref, ulin_ref, ilin_ref,
                  bias_ref, w1u_ref, w1i_ref, b1_ref,
                  h_ref, fmlin_ref, acc_ref):
    j = pl.program_id(0)
    ug = ug_ref[...]
    ig = ig_ref[...]
    u = jnp.where(upar_ref[...] > 0, ug[:, 64:], ug[:, :64])
    it = jnp.where(ipar_ref[...] > 0, ig[:, 64:], ig[:, :64])

    fm = jnp.sum(u * it, axis=1, keepdims=True)
    fmlin_ref[...] = fm + ulin_ref[...] + ilin_ref[...] + bias_ref[0, 0]

    h = (jnp.dot(u, w1u_ref[...], preferred_element_type=jnp.float32)
         + jnp.dot(it, w1i_ref[...], preferred_element_type=jnp.float32)
         + b1_ref[...])
    h_ref[...] = h

    @pl.when(j == 0)
    def _():
        acc_ref[...] = jnp.zeros_like(acc_ref)

    acc_ref[0:1, :] += jnp.sum(h, axis=0, keepdims=True)
    acc_ref[1:2, :] += jnp.sum(h * h, axis=0, keepdims=True)


def _tc_mlp2_body(h_ref, acc_ref, fmlin_ref, g1_ref, be1_ref,
                  w2_ref, b2_ref, g2_ref, be2_ref, w3_ref, b3_ref, out_ref):
    n = h_ref.shape[0]
    h = h_ref[...]
    m1 = acc_ref[0:1, :] * (1.0 / n)
    v1 = acc_ref[1:2, :] * (1.0 / n) - m1 * m1
    h = g1_ref[...] * (h - m1) * lax.rsqrt(v1 + _EPS) + be1_ref[...]
    h = jnp.maximum(h, 0.0)

    h2 = jnp.dot(h, w2_ref[...], preferred_element_type=jnp.float32) + b2_ref[...]
    m2 = jnp.mean(h2, axis=0, keepdims=True)
    v2 = jnp.mean((h2 - m2) * (h2 - m2), axis=0, keepdims=True)
    h2 = g2_ref[...] * (h2 - m2) * lax.rsqrt(v2 + _EPS) + be2_ref[...]
    h2 = jnp.maximum(h2, 0.0)

    deep = jnp.sum(h2 * w3_ref[...], axis=1, keepdims=True) + b3_ref[0, 0]
    out_ref[...] = jax.nn.sigmoid(fmlin_ref[...] + deep)


def _tc_mlp(ug, ig, upar, ipar, ulin, ilin, bias, W1, b1, g1, be1,
            W2, b2, g2, be2, W3, b3):
    batch = ug.shape[0]
    emb_dim = 64
    w1u = W1[:, :emb_dim].T
    w1i = W1[:, emb_dim:].T
    w2 = W2.T
    nb = batch // _M1_BLK

    col = lambda x: x.reshape(batch, 1)
    bcast = lambda j: (0, 0)
    h, fmlin, acc = pl.pallas_call(
        _tc_mlp1_body,
        grid=(nb,),
        in_specs=[
            pl.BlockSpec((_M1_BLK, 128), lambda j: (j, 0)),
            pl.BlockSpec((_M1_BLK, 128), lambda j: (j, 0)),
            pl.BlockSpec((_M1_BLK, 1), lambda j: (j, 0)),
            pl.BlockSpec((_M1_BLK, 1), lambda j: (j, 0)),
            pl.BlockSpec((_M1_BLK, 1), lambda j: (j, 0)),
            pl.BlockSpec((_M1_BLK, 1), lambda j: (j, 0)),
            pl.BlockSpec((1, 1), bcast),
            pl.BlockSpec((64, 128), bcast),
            pl.BlockSpec((64, 128), bcast),
            pl.BlockSpec((1, 128), bcast),
        ],
        out_specs=[
            pl.BlockSpec((_M1_BLK, 128), lambda j: (j, 0)),
            pl.BlockSpec((_M1_BLK, 1), lambda j: (j, 0)),
            pl.BlockSpec((2, 128), bcast),
        ],
        out_shape=[
            jax.ShapeDtypeStruct((batch, 128), jnp.float32),
            jax.ShapeDtypeStruct((batch, 1), jnp.float32),
            jax.ShapeDtypeStruct((2, 128), jnp.float32),
        ],
    )(ug, ig, col(upar), col(ipar), col(ulin), col(ilin), bias.reshape(1, 1),
      w1u, w1i, b1.reshape(1, -1))

    return pl.pallas_call(
        _tc_mlp2_body,
        out_shape=jax.ShapeDtypeStruct((batch, 1), jnp.float32),
        compiler_params=pltpu.CompilerParams(
            vmem_limit_bytes=60 * 1024 * 1024,
        ),
    )(h, acc, fmlin, g1.reshape(1, -1), be1.reshape(1, -1),
      w2, b2.reshape(1, -1), g2.reshape(1, -1), be2.reshape(1, -1),
      W3, b3.reshape(1, 1))


def kernel(user_ids, item_ids, user_emb_table, item_emb_table,
           user_lin_table, item_lin_table, bias,
           W1, b1, g1, be1, W2, b2, g2, be2, W3, b3):
    pu = _tc_pack(user_emb_table.T)
    pi = _tc_pack(item_emb_table.T)
    ulin_g, ilin_g = _sc_lin(user_ids, item_ids,
                             user_lin_table.reshape(-1),
                             item_lin_table.reshape(-1))
    half = _pack_half(user_emb_table.shape[0])
    upidx = jnp.where(user_ids < half, user_ids, user_ids - half)
    ipidx = jnp.where(item_ids < half, item_ids, item_ids - half)
    ug, ig = _sc_gather_packed(upidx, ipidx, pu, pi)
    upar = (user_ids >= half).astype(jnp.int32)
    ipar = (item_ids >= half).astype(jnp.int32)
    pred = _tc_mlp(ug, ig, upar, ipar, ulin_g, ilin_g,
                   bias, W1, b1, g1, be1, W2, b2, g2, be2, W3, b3)
    return jnp.squeeze(pred, axis=-1)
